# baseline (speedup 1.0000x reference)
import functools

import jax
import jax.numpy as jnp
import numpy as np
from jax import lax
from jax.experimental import pallas as pl
from jax.experimental.pallas import tpu as pltpu
from jax.experimental.pallas import tpu_sc as plsc

C = 16
D = 512
DIN = 256
DOUT = 256
EPS = 1e-20

_NC = 2
_L = 16


def _uniform_noise(seed: int, n: int) -> np.ndarray:
    k0, k1 = np.uint32(seed >> 32), np.uint32(seed & 0xFFFFFFFF)
    ks = [k0, k1, np.uint32(k0 ^ k1 ^ np.uint32(0x1BD11BDA))]
    x0 = np.zeros(n, dtype=np.uint32)
    x1 = np.arange(n, dtype=np.uint32)

    def rotl(v, r):
        return (v << r) | (v >> np.uint32(32 - r))

    x0 += ks[0]
    x1 += ks[1]
    rot = [np.uint32(r) for r in (13, 15, 26, 6, 17, 29, 16, 24)]
    order = [(1, 2), (2, 0), (0, 1), (1, 2), (2, 0)]
    for g in range(5):
        for r in rot[0:4] if g % 2 == 0 else rot[4:8]:
            x0 += x1
            x1 = rotl(x1, r)
            x1 ^= x0
        a, b = order[g]
        x0 += ks[a]
        x1 += ks[b] + np.uint32(g + 1)
    bits = x0 ^ x1
    fl = ((bits >> np.uint32(9)) | np.uint32(0x3F800000)).view(np.float32) - 1.0
    return np.maximum(0.0, fl).astype(np.float32)


_NOISE = _uniform_noise(42, C * D).reshape(C, 1, D)


_CB = 8
_NSTEP = C // _CB


def _tc_logits_argmax(x_ref, w_ref, b_ref, noise_ref, knn_ref, idx_ref, v_ref):
    c = pl.program_id(0)
    logits = jnp.dot(x_ref[0:1], w_ref[...],
                     preferred_element_type=jnp.float32) + b_ref[...]
    z = logits.reshape(_CB, D)
    scale = knn_ref[0]
    samples = scale * (noise_ref[:, 0, :] - 0.5) + 0.5
    g = -jnp.log(-jnp.log(samples + EPS) + EPS)
    v_ref[c] = z + g

    @pl.when(c == _NSTEP - 1)
    def _():
        ind = jnp.argmax(v_ref[...].reshape(C, D), axis=-1).astype(jnp.int32)
        idx_ref[...] = (ind + lax.iota(jnp.int32, C) * D).reshape(1, C)


def _sc_decode(idx_hbm, wdec_hbm, bdec_hbm, out_hbm, idx_v, rows_v, acc_v,
               semi, semb, semg):
    ci = pltpu.async_copy(idx_hbm, idx_v, semi)
    cb = pltpu.async_copy(bdec_hbm, acc_v, semb)
    ci.wait()
    cg = pltpu.async_copy(wdec_hbm.at[idx_v], rows_v, semg)
    cb.wait()
    cg.wait()
    for j in range(DOUT // _L):
        sl = pl.ds(j * _L, _L)
        acc = acc_v[sl]
        for r in range(C):
            acc = acc + rows_v[r, sl]
        acc_v[sl] = acc
    pltpu.sync_copy(acc_v, out_hbm)


def kernel(x, knn, W_est, b_est, W_dec, b_dec):
    idx = pl.pallas_call(
        _tc_logits_argmax,
        grid=(_NSTEP,),
        out_shape=jax.ShapeDtypeStruct((1, C), jnp.int32),
        in_specs=[
            pl.BlockSpec((8, DIN), lambda c: (0, 0)),
            pl.BlockSpec((DIN, _CB * D), lambda c: (0, c)),
            pl.BlockSpec((1, _CB * D), lambda c: (0, c)),
            pl.BlockSpec((_CB, 1, D), lambda c: (c, 0, 0)),
            pl.BlockSpec(memory_space=pltpu.SMEM),
        ],
        out_specs=pl.BlockSpec((1, C), lambda c: (0, 0)),
        scratch_shapes=[pltpu.VMEM((_NSTEP, _CB, D), jnp.float32)],
    )(x, W_est, b_est.reshape(1, C * D), jnp.asarray(_NOISE), knn[:1])

    mesh = plsc.VectorSubcoreMesh(core_axis_name="c", subcore_axis_name="s",
                                  num_cores=1, num_subcores=1)
    decode = functools.partial(
        pl.kernel,
        out_type=jax.ShapeDtypeStruct((DOUT,), jnp.float32),
        mesh=mesh,
        scratch_types=[
            pltpu.VMEM((C,), jnp.int32),
            pltpu.VMEM((C, DOUT), jnp.float32),
            pltpu.VMEM((DOUT,), jnp.float32),
            pltpu.SemaphoreType.DMA,
            pltpu.SemaphoreType.DMA,
            pltpu.SemaphoreType.DMA,
        ],
    )(_sc_decode)
    out = decode(idx.reshape(C), W_dec, b_dec)

    def _copy(i_ref, o_ref):
        o_ref[...] = i_ref[...]

    out = pl.pallas_call(
        _copy,
        out_shape=jax.ShapeDtypeStruct((1, DOUT), jnp.float32),
        in_specs=[pl.BlockSpec(memory_space=pltpu.VMEM)],
        out_specs=pl.BlockSpec(memory_space=pltpu.VMEM),
    )(out.reshape(1, DOUT))
    return out.reshape(DOUT)


# speedup vs baseline: 1.0547x; 1.0547x over previous
import functools

import jax
import jax.numpy as jnp
import numpy as np
from jax import lax
from jax.experimental import pallas as pl
from jax.experimental.pallas import tpu as pltpu
from jax.experimental.pallas import tpu_sc as plsc

C = 16
D = 512
DIN = 256
DOUT = 256
EPS = 1e-20

_NC = 2
_L = 16


def _uniform_noise(seed: int, n: int) -> np.ndarray:
    k0, k1 = np.uint32(seed >> 32), np.uint32(seed & 0xFFFFFFFF)
    ks = [k0, k1, np.uint32(k0 ^ k1 ^ np.uint32(0x1BD11BDA))]
    x0 = np.zeros(n, dtype=np.uint32)
    x1 = np.arange(n, dtype=np.uint32)

    def rotl(v, r):
        return (v << r) | (v >> np.uint32(32 - r))

    x0 += ks[0]
    x1 += ks[1]
    rot = [np.uint32(r) for r in (13, 15, 26, 6, 17, 29, 16, 24)]
    order = [(1, 2), (2, 0), (0, 1), (1, 2), (2, 0)]
    for g in range(5):
        for r in rot[0:4] if g % 2 == 0 else rot[4:8]:
            x0 += x1
            x1 = rotl(x1, r)
            x1 ^= x0
        a, b = order[g]
        x0 += ks[a]
        x1 += ks[b] + np.uint32(g + 1)
    bits = x0 ^ x1
    fl = ((bits >> np.uint32(9)) | np.uint32(0x3F800000)).view(np.float32) - 1.0
    return np.maximum(0.0, fl).astype(np.float32)


_NOISE = _uniform_noise(42, C * D).reshape(C, 1, D)


_CB = 8
_NSTEP = C // _CB


def _tc_logits_argmax(x_ref, w_ref, b_ref, noise_ref, knn_ref, idx_ref, v_ref):
    c = pl.program_id(0)
    logits = jnp.dot(x_ref[0:1], w_ref[...],
                     preferred_element_type=jnp.float32) + b_ref[...]
    z = logits.reshape(_CB, D)
    scale = knn_ref[0]
    samples = scale * (noise_ref[:, 0, :] - 0.5) + 0.5
    g = -jnp.log(-jnp.log(samples + EPS) + EPS)
    v_ref[c] = z + g

    @pl.when(c == _NSTEP - 1)
    def _():
        ind = jnp.argmax(v_ref[...].reshape(C, D), axis=-1).astype(jnp.int32)
        idx_ref[...] = (ind + lax.iota(jnp.int32, C) * D).reshape(1, C)


def _sc_decode(idx_hbm, wdec_hbm, bdec_hbm, out_hbm, idx_v, rows_v, acc_v,
               semi, semb, semg):
    ci = pltpu.async_copy(idx_hbm, idx_v, semi)
    cb = pltpu.async_copy(bdec_hbm, acc_v, semb)
    ci.wait()
    cg = pltpu.async_copy(wdec_hbm.at[idx_v], rows_v, semg)
    cb.wait()
    cg.wait()
    for j in range(DOUT // _L):
        sl = pl.ds(j * _L, _L)
        acc = acc_v[sl]
        for r in range(C):
            acc = acc + rows_v[r, sl]
        acc_v[sl] = acc
    pltpu.sync_copy(acc_v, out_hbm)


def kernel(x, knn, W_est, b_est, W_dec, b_dec):
    idx = pl.pallas_call(
        _tc_logits_argmax,
        grid=(_NSTEP,),
        out_shape=jax.ShapeDtypeStruct((1, C), jnp.int32),
        in_specs=[
            pl.BlockSpec((8, DIN), lambda c: (0, 0)),
            pl.BlockSpec((DIN, _CB * D), lambda c: (0, c)),
            pl.BlockSpec((1, _CB * D), lambda c: (0, c)),
            pl.BlockSpec((_CB, 1, D), lambda c: (c, 0, 0)),
            pl.BlockSpec(memory_space=pltpu.SMEM),
        ],
        out_specs=pl.BlockSpec((1, C), lambda c: (0, 0)),
        scratch_shapes=[pltpu.VMEM((_NSTEP, _CB, D), jnp.float32)],
    )(x, W_est, b_est.reshape(1, C * D), jnp.asarray(_NOISE), knn[:1])

    mesh = plsc.VectorSubcoreMesh(core_axis_name="c", subcore_axis_name="s",
                                  num_cores=1, num_subcores=1)
    decode = functools.partial(
        pl.kernel,
        out_type=jax.ShapeDtypeStruct((DOUT,), jnp.float32),
        mesh=mesh,
        scratch_types=[
            pltpu.VMEM((C,), jnp.int32),
            pltpu.VMEM((C, DOUT), jnp.float32),
            pltpu.VMEM((DOUT,), jnp.float32),
            pltpu.SemaphoreType.DMA,
            pltpu.SemaphoreType.DMA,
            pltpu.SemaphoreType.DMA,
        ],
    )(_sc_decode)
    return decode(idx.reshape(C), W_dec, b_dec)


# speedup vs baseline: 1.0594x; 1.0045x over previous
import functools

import jax
import jax.numpy as jnp
import numpy as np
from jax import lax
from jax.experimental import pallas as pl
from jax.experimental.pallas import tpu as pltpu
from jax.experimental.pallas import tpu_sc as plsc

C = 16
D = 512
DIN = 256
DOUT = 256
EPS = 1e-20

_L = 16


def _uniform_noise(seed: int, n: int) -> np.ndarray:
    k0, k1 = np.uint32(seed >> 32), np.uint32(seed & 0xFFFFFFFF)
    ks = [k0, k1, np.uint32(k0 ^ k1 ^ np.uint32(0x1BD11BDA))]
    x0 = np.zeros(n, dtype=np.uint32)
    x1 = np.arange(n, dtype=np.uint32)

    def rotl(v, r):
        return (v << r) | (v >> np.uint32(32 - r))

    x0 += ks[0]
    x1 += ks[1]
    rot = [np.uint32(r) for r in (13, 15, 26, 6, 17, 29, 16, 24)]
    order = [(1, 2), (2, 0), (0, 1), (1, 2), (2, 0)]
    for g in range(5):
        for r in rot[0:4] if g % 2 == 0 else rot[4:8]:
            x0 += x1
            x1 = rotl(x1, r)
            x1 ^= x0
        a, b = order[g]
        x0 += ks[a]
        x1 += ks[b] + np.uint32(g + 1)
    bits = x0 ^ x1
    fl = ((bits >> np.uint32(9)) | np.uint32(0x3F800000)).view(np.float32) - 1.0
    return np.maximum(0.0, fl).astype(np.float32)


_NOISE = _uniform_noise(42, C * D).reshape(C, 1, D)


_CB = 8
_NSTEP = C // _CB


def _tc_logits_argmax(x_ref, w_ref, b_ref, noise_ref, knn_ref, idx_ref, v_ref):
    c = pl.program_id(0)
    logits = jnp.dot(x_ref[0:1], w_ref[...],
                     preferred_element_type=jnp.float32) + b_ref[...]
    z = logits.reshape(_CB, D)
    scale = knn_ref[0]
    samples = scale * (noise_ref[:, 0, :] - 0.5) + 0.5
    g = -jnp.log(-jnp.log(samples + EPS) + EPS)
    v_ref[c] = z + g

    @pl.when(c == _NSTEP - 1)
    def _():
        ind = jnp.argmax(v_ref[...].reshape(C, D), axis=-1).astype(jnp.int32)
        idx_ref[...] = (ind + lax.iota(jnp.int32, C) * D).reshape(1, C)


def _sc_decode(idx_hbm, wdec_hbm, bdec_hbm, out_hbm, idx_v, rows_v, acc_v,
               semi, semb, semg):
    ci = pltpu.async_copy(idx_hbm, idx_v, semi)
    cb = pltpu.async_copy(bdec_hbm, acc_v, semb)
    ci.wait()
    cg = pltpu.async_copy(wdec_hbm.at[idx_v], rows_v, semg)
    cb.wait()
    cg.wait()
    for j in range(DOUT // _L):
        sl = pl.ds(j * _L, _L)
        acc = acc_v[sl]
        for r in range(C):
            acc = acc + rows_v[r, sl]
        acc_v[sl] = acc
    pltpu.sync_copy(acc_v, out_hbm)


def kernel(x, knn, W_est, b_est, W_dec, b_dec):
    idx = pl.pallas_call(
        _tc_logits_argmax,
        grid=(_NSTEP,),
        out_shape=jax.ShapeDtypeStruct((1, C), jnp.int32),
        in_specs=[
            pl.BlockSpec((8, DIN), lambda c: (0, 0)),
            pl.BlockSpec((DIN, _CB * D), lambda c: (0, c)),
            pl.BlockSpec((1, _CB * D), lambda c: (0, c)),
            pl.BlockSpec((_CB, 1, D), lambda c: (c, 0, 0)),
            pl.BlockSpec(memory_space=pltpu.SMEM),
        ],
        out_specs=pl.BlockSpec((1, C), lambda c: (0, 0)),
        scratch_shapes=[pltpu.VMEM((_NSTEP, _CB, D), jnp.float32)],
    )(x, W_est, b_est.reshape(1, C * D), jnp.asarray(_NOISE), knn[:1])

    mesh = plsc.VectorSubcoreMesh(core_axis_name="c", subcore_axis_name="s",
                                  num_cores=1, num_subcores=1)
    decode = functools.partial(
        pl.kernel,
        out_type=jax.ShapeDtypeStruct((DOUT,), jnp.float32),
        mesh=mesh,
        scratch_types=[
            pltpu.VMEM((C,), jnp.int32),
            pltpu.VMEM((C, DOUT), jnp.float32),
            pltpu.VMEM((DOUT,), jnp.float32),
            pltpu.SemaphoreType.DMA,
            pltpu.SemaphoreType.DMA,
            pltpu.SemaphoreType.DMA,
        ],
    )(_sc_decode)
    return decode(idx.reshape(C), W_dec, b_dec)
